# Initial kernel scaffold; baseline (speedup 1.0000x reference)
import jax
import jax.numpy as jnp
from jax.experimental import pallas as pl


def kernel(band_ids, bias):
    raise NotImplementedError("write your pallas kernel here")


# speedup vs baseline: 8.6812x; 8.6812x over previous
import functools

import jax
import jax.numpy as jnp
from jax import lax
from jax.experimental import pallas as pl
from jax.experimental.pallas import tpu as pltpu
from jax.experimental.pallas import tpu_sc as plsc

H = 8
NC = 2
NS = 16
NW = NC * NS
CHUNK = 4000


def _sc_lookup(e_total):
    per_w = e_total // NW
    n_chunks = per_w // CHUNK
    mesh = plsc.VectorSubcoreMesh(core_axis_name="c", subcore_axis_name="s")

    @functools.partial(
        pl.kernel,
        out_type=jax.ShapeDtypeStruct((e_total * H,), jnp.float32),
        mesh=mesh,
        compiler_params=pltpu.CompilerParams(needs_layout_passes=False),
        scratch_types=[
            pltpu.VMEM((CHUNK,), jnp.int32),
            pltpu.VMEM((CHUNK,), jnp.int32),
            pltpu.VMEM((CHUNK * H,), jnp.float32),
            pltpu.VMEM((CHUNK * H,), jnp.float32),
            pltpu.VMEM((48,), jnp.float32),
            pltpu.SemaphoreType.DMA,
            pltpu.SemaphoreType.DMA,
            pltpu.SemaphoreType.DMA,
            pltpu.SemaphoreType.DMA,
        ],
    )
    def body(ids_hbm, bias_hbm, out_hbm, ids_v0, ids_v1, out_v0, out_v1,
             bias_v, in_sem0, in_sem1, out_sem0, out_sem1):
        c = lax.axis_index("c")
        s = lax.axis_index("s")
        wid = s * NC + c
        base = wid * per_w
        in_sems = (in_sem0, in_sem1)
        out_sems = (out_sem0, out_sem1)
        ids_bufs = (ids_v0, ids_v1)
        out_bufs = (out_v0, out_v1)
        pltpu.sync_copy(bias_hbm, bias_v)
        iota = lax.iota(jnp.int32, 16)
        iota8 = iota * H

        def ids_copy(ci, b):
            return pltpu.make_async_copy(
                ids_hbm.at[pl.ds(base + ci * CHUNK, CHUNK)],
                ids_bufs[b], in_sems[b])

        def out_copy(ci, b):
            return pltpu.make_async_copy(
                out_bufs[b],
                out_hbm.at[pl.ds((base + ci * CHUNK) * H, CHUNK * H)],
                out_sems[b])

        ids_copy(0, 0).start()
        ids_copy(1, 1).start()

        def two_chunks(i, carry):
            for b in range(2):
                ci = i * 2 + b
                ids_copy(ci, b).wait()
                @pl.when(ci >= 2)
                def _():
                    out_copy(ci - 2, b).wait()

                idsb = ids_bufs[b]
                outb = out_bufs[b]

                @plsc.parallel_loop(0, CHUNK // 16, unroll=4)
                def _(k):
                    v8 = idsb[pl.ds(k * 16, 16)] * H
                    st = iota8 + k * 128
                    for h in range(H):
                        col = plsc.load_gather(bias_v, [v8 + h])
                        plsc.store_scatter(outb, [st + h], col)

                out_copy(ci, b).start()

                @pl.when(ci + 2 < n_chunks)
                def _():
                    ids_copy(ci + 2, b).start()
            return carry

        lax.fori_loop(0, n_chunks // 2, two_chunks, 0)
        out_copy(n_chunks - 2, 0).wait()
        out_copy(n_chunks - 1, 1).wait()

    return body


def kernel(band_ids, bias):
    e_total = band_ids.shape[0]
    ids = band_ids.astype(jnp.int32)
    bias_flat = jnp.pad(bias.reshape(-1).astype(jnp.float32), (0, 8))
    out_flat = _sc_lookup(e_total)(ids, bias_flat)
    return out_flat.reshape(e_total, H)


# speedup vs baseline: 8.7073x; 1.0030x over previous
import functools

import jax
import jax.numpy as jnp
from jax import lax
from jax.experimental import pallas as pl
from jax.experimental.pallas import tpu as pltpu
from jax.experimental.pallas import tpu_sc as plsc

H = 8
NC = 2
NS = 16
NW = NC * NS
CHUNK = 4000


def _sc_lookup(e_total):
    per_w = e_total // NW
    n_chunks = per_w // CHUNK
    mesh = plsc.VectorSubcoreMesh(core_axis_name="c", subcore_axis_name="s")

    @functools.partial(
        pl.kernel,
        out_type=jax.ShapeDtypeStruct((e_total, H), jnp.float32),
        mesh=mesh,
        compiler_params=pltpu.CompilerParams(needs_layout_passes=False, use_tc_tiling_on_sc=False),
        scratch_types=[
            pltpu.VMEM((CHUNK,), jnp.int32),
            pltpu.VMEM((CHUNK,), jnp.int32),
            pltpu.VMEM((CHUNK, H), jnp.float32),
            pltpu.VMEM((CHUNK, H), jnp.float32),
            pltpu.VMEM((48,), jnp.float32),
            pltpu.SemaphoreType.DMA,
            pltpu.SemaphoreType.DMA,
            pltpu.SemaphoreType.DMA,
            pltpu.SemaphoreType.DMA,
        ],
    )
    def body(ids_hbm, bias_hbm, out_hbm, ids_v0, ids_v1, out_v0, out_v1,
             bias_v, in_sem0, in_sem1, out_sem0, out_sem1):
        c = lax.axis_index("c")
        s = lax.axis_index("s")
        wid = s * NC + c
        base = wid * per_w
        in_sems = (in_sem0, in_sem1)
        out_sems = (out_sem0, out_sem1)
        ids_bufs = (ids_v0, ids_v1)
        out_bufs = (out_v0, out_v1)
        pltpu.sync_copy(bias_hbm, bias_v)
        iota = lax.iota(jnp.int32, 16)

        def ids_copy(ci, b):
            return pltpu.make_async_copy(
                ids_hbm.at[pl.ds(base + ci * CHUNK, CHUNK)],
                ids_bufs[b], in_sems[b])

        def out_copy(ci, b):
            return pltpu.make_async_copy(
                out_bufs[b],
                out_hbm.at[pl.ds(base + ci * CHUNK, CHUNK)],
                out_sems[b])

        ids_copy(0, 0).start()
        ids_copy(1, 1).start()

        def two_chunks(i, carry):
            for b in range(2):
                ci = i * 2 + b
                ids_copy(ci, b).wait()
                @pl.when(ci >= 2)
                def _():
                    out_copy(ci - 2, b).wait()

                idsb = ids_bufs[b]
                outb = out_bufs[b]

                @plsc.parallel_loop(0, CHUNK // 16, unroll=4)
                def _(k):
                    rows = iota + k * 16
                    v8 = idsb[pl.ds(k * 16, 16)] * H
                    for h in range(H):
                        col = plsc.load_gather(bias_v, [v8 + h])
                        plsc.store_scatter(
                            outb, [rows, jnp.full((16,), h, jnp.int32)], col)

                out_copy(ci, b).start()

                @pl.when(ci + 2 < n_chunks)
                def _():
                    ids_copy(ci + 2, b).start()
            return carry

        lax.fori_loop(0, n_chunks // 2, two_chunks, 0)
        out_copy(n_chunks - 2, 0).wait()
        out_copy(n_chunks - 1, 1).wait()

    return body


def kernel(band_ids, bias):
    e_total = band_ids.shape[0]
    ids = band_ids.astype(jnp.int32)
    bias_flat = jnp.pad(bias.reshape(-1).astype(jnp.float32), (0, 8))
    return _sc_lookup(e_total)(ids, bias_flat)


# speedup vs baseline: 24.5341x; 2.8176x over previous
import functools

import jax
import jax.numpy as jnp
from jax import lax
from jax.experimental import pallas as pl
from jax.experimental.pallas import tpu as pltpu
from jax.experimental.pallas import tpu_sc as plsc

H = 8
NC = 2
NS = 16
NW = NC * NS
CHUNK = 4000


def _sc_lookup(e_total):
    per_w = e_total // NW
    n_chunks = per_w // CHUNK
    mesh = plsc.VectorSubcoreMesh(core_axis_name="c", subcore_axis_name="s")

    @functools.partial(
        pl.kernel,
        out_type=jax.ShapeDtypeStruct((e_total, 128), jnp.float32),
        mesh=mesh,
        compiler_params=pltpu.CompilerParams(needs_layout_passes=False, use_tc_tiling_on_sc=False),
        scratch_types=[
            pltpu.VMEM((CHUNK,), jnp.int32),
            pltpu.VMEM((CHUNK,), jnp.int32),
            pltpu.VMEM((CHUNK, H), jnp.float32),
            pltpu.VMEM((CHUNK, H), jnp.float32),
            pltpu.VMEM((48,), jnp.float32),
            pltpu.SemaphoreType.DMA,
            pltpu.SemaphoreType.DMA,
            pltpu.SemaphoreType.DMA,
            pltpu.SemaphoreType.DMA,
        ],
    )
    def body(ids_hbm, bias_hbm, out_hbm, ids_v0, ids_v1, out_v0, out_v1,
             bias_v, in_sem0, in_sem1, out_sem0, out_sem1):
        c = lax.axis_index("c")
        s = lax.axis_index("s")
        wid = s * NC + c
        base = wid * per_w
        in_sems = (in_sem0, in_sem1)
        out_sems = (out_sem0, out_sem1)
        ids_bufs = (ids_v0, ids_v1)
        out_bufs = (out_v0, out_v1)
        pltpu.sync_copy(bias_hbm, bias_v)
        iota = lax.iota(jnp.int32, 16)

        def ids_copy(ci, b):
            return pltpu.make_async_copy(
                ids_hbm.at[pl.ds(base + ci * CHUNK, CHUNK)],
                ids_bufs[b], in_sems[b])

        def out_copy(ci, b):
            return pltpu.make_async_copy(
                out_bufs[b],
                out_hbm.at[pl.ds(base + ci * CHUNK, CHUNK), pl.ds(0, H)],
                out_sems[b])

        ids_copy(0, 0).start()
        ids_copy(1, 1).start()

        def two_chunks(i, carry):
            for b in range(2):
                ci = i * 2 + b
                ids_copy(ci, b).wait()
                @pl.when(ci >= 2)
                def _():
                    out_copy(ci - 2, b).wait()

                idsb = ids_bufs[b]
                outb = out_bufs[b]

                @plsc.parallel_loop(0, CHUNK // 16, unroll=4)
                def _(k):
                    rows = iota + k * 16
                    v8 = idsb[pl.ds(k * 16, 16)] * H
                    for h in range(H):
                        col = plsc.load_gather(bias_v, [v8 + h])
                        plsc.store_scatter(
                            outb, [rows, jnp.full((16,), h, jnp.int32)], col)

                out_copy(ci, b).start()

                @pl.when(ci + 2 < n_chunks)
                def _():
                    ids_copy(ci + 2, b).start()
            return carry

        lax.fori_loop(0, n_chunks // 2, two_chunks, 0)
        out_copy(n_chunks - 2, 0).wait()
        out_copy(n_chunks - 1, 1).wait()

    return body


def kernel(band_ids, bias):
    e_total = band_ids.shape[0]
    ids = band_ids.astype(jnp.int32)
    bias_flat = jnp.pad(bias.reshape(-1).astype(jnp.float32), (0, 8))
    out_wide = _sc_lookup(e_total)(ids, bias_flat)
    return out_wide[:, :H]
